# Initial kernel scaffold; baseline (speedup 1.0000x reference)
import jax
import jax.numpy as jnp
from jax.experimental import pallas as pl


def kernel(y_hat, y, weights):
    raise NotImplementedError("write your pallas kernel here")


# speedup vs baseline: 3.0463x; 3.0463x over previous
import jax
import jax.numpy as jnp
from jax import lax
from jax.experimental import pallas as pl

_B = 16384
_C = 1000
_K = 11468
_LMBDA = 0.5
_RB = 256
_NG = _B // _RB


def _row_stats_kernel(w_ref, yh_ref, y_ref, s_ref, l_ref, wr_ref):
    y = y_ref[...]
    yh = yh_ref[...]
    m = jnp.max(y, axis=1)
    col = lax.broadcasted_iota(jnp.int32, (_RB, _C), 1)
    t = jnp.min(jnp.where(y == m[:, None], col, _C), axis=1)
    hit = col == t[:, None]
    v = jnp.sum(jnp.where(hit, yh, 0.0), axis=1)
    wrow = jnp.sum(jnp.where(hit, w_ref[...], 0.0), axis=1)
    d = jnp.sum(yh * y, axis=1)
    s_ref[0, 0, :] = -v + _LMBDA * (1.0 - d)
    l_ref[0, 0, :] = -jnp.log(v)
    wr_ref[0, 0, :] = wrow


def _select_kernel(s_ref, l_ref, w_ref, out_ref):
    s = s_ref[...]
    b = lax.bitcast_convert_type(s, jnp.int32)
    keys = jnp.where(b < 0,
                     jnp.bitwise_xor(jnp.bitwise_not(b), jnp.int32(-2147483648)),
                     b)

    def count_ge(thr):
        return jnp.sum((keys >= thr).astype(jnp.int32))

    def body(_, carry):
        lo, hi = carry
        d = hi - lo
        half = lax.shift_right_logical(d, 1) + jnp.bitwise_and(d, 1)
        mid = lo + half
        cond = count_ge(mid) >= _K
        return (jnp.where(cond, mid, lo), jnp.where(cond, hi, mid - 1))

    lo, _ = lax.fori_loop(0, 32, body,
                          (jnp.int32(-2147483648), jnp.int32(2147483647)))
    tau = lo
    gt = keys > tau
    eq = keys == tau
    n_gt = jnp.sum(gt.astype(jnp.int32))
    need = _K - n_gt

    lin = (lax.broadcasted_iota(jnp.int32, (128, 128), 0) * 128
           + lax.broadcasted_iota(jnp.int32, (128, 128), 1))

    def body2(_, carry):
        lo2, hi2 = carry
        mid = lax.shift_right_logical(lo2 + hi2, 1)
        cnt = jnp.sum((eq & (lin <= mid)).astype(jnp.int32))
        cond = cnt >= need
        return (jnp.where(cond, lo2, mid + 1), jnp.where(cond, mid, hi2))

    c, _ = lax.fori_loop(0, 14, body2, (jnp.int32(0), jnp.int32(_B - 1)))
    sel = gt | (eq & (lin <= c))

    w = w_ref[...]
    l = l_ref[...]
    num = jnp.sum(jnp.where(sel, w * l, 0.0))
    den = jnp.sum(jnp.where(sel, w, 0.0))
    out_ref[...] = jnp.broadcast_to(num / den, (1, 128))


def kernel(y_hat, y, weights):
    w2 = weights.reshape(1, _C)
    s, l, wr = pl.pallas_call(
        _row_stats_kernel,
        grid=(_NG,),
        in_specs=[
            pl.BlockSpec((1, _C), lambda g: (0, 0)),
            pl.BlockSpec((_RB, _C), lambda g: (g, 0)),
            pl.BlockSpec((_RB, _C), lambda g: (g, 0)),
        ],
        out_specs=[
            pl.BlockSpec((1, 1, _RB), lambda g: (g, 0, 0)),
            pl.BlockSpec((1, 1, _RB), lambda g: (g, 0, 0)),
            pl.BlockSpec((1, 1, _RB), lambda g: (g, 0, 0)),
        ],
        out_shape=[
            jax.ShapeDtypeStruct((_NG, 1, _RB), jnp.float32),
            jax.ShapeDtypeStruct((_NG, 1, _RB), jnp.float32),
            jax.ShapeDtypeStruct((_NG, 1, _RB), jnp.float32),
        ],
    )(w2, y_hat, y)

    out = pl.pallas_call(
        _select_kernel,
        out_shape=jax.ShapeDtypeStruct((1, 128), jnp.float32),
    )(s.reshape(128, 128), l.reshape(128, 128), wr.reshape(128, 128))
    return out[0, 0]


# speedup vs baseline: 9.6293x; 3.1610x over previous
import jax
import jax.numpy as jnp
from jax import lax
from jax.experimental import pallas as pl
from jax.experimental.pallas import tpu as pltpu

_B = 16384
_C = 1000
_K = 11468
_LMBDA = 0.5
_CB = 512
_NG = _B // _CB


def _row_stats_kernel(w_ref, yh_ref, y_ref, s_ref, l_ref, wr_ref):
    yv = y_ref[...]
    yh = yh_ref[...]
    m = jnp.max(yv, axis=0)
    row = lax.broadcasted_iota(jnp.int32, (_C, _CB), 0)
    t = jnp.min(jnp.where(yv == m[None, :], row, _C), axis=0)
    hit = row == t[None, :]
    v = jnp.sum(jnp.where(hit, yh, 0.0), axis=0)
    wrow = jnp.sum(jnp.where(hit, w_ref[...], 0.0), axis=0)
    d = jnp.sum(yh * yv, axis=0)
    s_ref[0, :] = -v + _LMBDA * (1.0 - d)
    l_ref[0, :] = -jnp.log(v)
    wr_ref[0, :] = wrow


def _select_kernel(s_ref, l_ref, w_ref, out_ref):
    s = s_ref[...]
    b = lax.bitcast_convert_type(s, jnp.int32)
    keys = jnp.where(b < 0,
                     jnp.bitwise_xor(jnp.bitwise_not(b), jnp.int32(-2147483648)),
                     b)

    def count_ge(thr):
        return jnp.sum((keys >= thr).astype(jnp.int32))

    def body(_, carry):
        lo, hi = carry
        d = hi - lo
        half = lax.shift_right_logical(d, 1) + jnp.bitwise_and(d, 1)
        mid = lo + half
        cond = count_ge(mid) >= _K
        return (jnp.where(cond, mid, lo), jnp.where(cond, hi, mid - 1))

    lo, _ = lax.fori_loop(0, 32, body,
                          (jnp.int32(-2147483648), jnp.int32(2147483647)))
    tau = lo
    gt = keys > tau
    eq = keys == tau
    n_gt = jnp.sum(gt.astype(jnp.int32))
    need = _K - n_gt

    lin = (lax.broadcasted_iota(jnp.int32, (128, 128), 0) * 128
           + lax.broadcasted_iota(jnp.int32, (128, 128), 1))

    def body2(_, carry):
        lo2, hi2 = carry
        mid = lax.shift_right_logical(lo2 + hi2, 1)
        cnt = jnp.sum((eq & (lin <= mid)).astype(jnp.int32))
        cond = cnt >= need
        return (jnp.where(cond, lo2, mid + 1), jnp.where(cond, mid, hi2))

    c, _ = lax.fori_loop(0, 14, body2, (jnp.int32(0), jnp.int32(_B - 1)))
    sel = gt | (eq & (lin <= c))

    w = w_ref[...]
    l = l_ref[...]
    num = jnp.sum(jnp.where(sel, w * l, 0.0))
    den = jnp.sum(jnp.where(sel, w, 0.0))
    out_ref[...] = jnp.broadcast_to(num / den, (1, 128))


def kernel(y_hat, y, weights):
    yh_t = y_hat.T
    y_t = y.T
    w2 = weights.reshape(_C, 1)
    s, l, wr = pl.pallas_call(
        _row_stats_kernel,
        grid=(_NG,),
        in_specs=[
            pl.BlockSpec((_C, 1), lambda g: (0, 0)),
            pl.BlockSpec((_C, _CB), lambda g: (0, g)),
            pl.BlockSpec((_C, _CB), lambda g: (0, g)),
        ],
        out_specs=[
            pl.BlockSpec((1, _CB), lambda g: (0, g)),
            pl.BlockSpec((1, _CB), lambda g: (0, g)),
            pl.BlockSpec((1, _CB), lambda g: (0, g)),
        ],
        out_shape=[
            jax.ShapeDtypeStruct((1, _B), jnp.float32),
            jax.ShapeDtypeStruct((1, _B), jnp.float32),
            jax.ShapeDtypeStruct((1, _B), jnp.float32),
        ],
    )(w2, yh_t, y_t)

    out = pl.pallas_call(
        _select_kernel,
        out_shape=jax.ShapeDtypeStruct((1, 128), jnp.float32),
    )(s.reshape(128, 128), l.reshape(128, 128), wr.reshape(128, 128))
    return out[0, 0]


# speedup vs baseline: 10.2650x; 1.0660x over previous
import jax
import jax.numpy as jnp
from jax import lax
from jax.experimental import pallas as pl
from jax.experimental.pallas import tpu as pltpu

_B = 16384
_C = 1000
_K = 11468
_LMBDA = 0.5
_CB = 1024
_NG = _B // _CB


def _row_stats_kernel(w_ref, yh_ref, y_ref, o_ref):
    yv = y_ref[...]
    yh = yh_ref[...]
    m = jnp.max(yv, axis=0)
    row = lax.broadcasted_iota(jnp.int32, (_C, _CB), 0)
    t = jnp.min(jnp.where(yv == m[None, :], row, _C), axis=0)
    hit = row == t[None, :]
    v = jnp.sum(jnp.where(hit, yh, 0.0), axis=0)
    wrow = jnp.sum(jnp.where(hit, w_ref[...], 0.0), axis=0)
    d = jnp.sum(yh * yv, axis=0)
    o_ref[0, :] = -v + _LMBDA * (1.0 - d)
    o_ref[1, :] = -jnp.log(v)
    o_ref[2, :] = wrow


def _select_kernel(s_ref, l_ref, w_ref, out_ref):
    s = s_ref[...]
    b = lax.bitcast_convert_type(s, jnp.int32)
    keys = jnp.where(b < 0,
                     jnp.bitwise_xor(jnp.bitwise_not(b), jnp.int32(-2147483648)),
                     b)

    def count_ge(thr):
        return jnp.sum((keys >= thr).astype(jnp.int32))

    def body(_, carry):
        lo, hi = carry
        d = hi - lo
        half = lax.shift_right_logical(d, 1) + jnp.bitwise_and(d, 1)
        mid = lo + half
        cond = count_ge(mid) >= _K
        return (jnp.where(cond, mid, lo), jnp.where(cond, hi, mid - 1))

    lo, _ = lax.fori_loop(0, 32, body,
                          (jnp.int32(-2147483648), jnp.int32(2147483647)))
    tau = lo
    gt = keys > tau
    eq = keys == tau
    n_gt = jnp.sum(gt.astype(jnp.int32))
    need = _K - n_gt

    lin = (lax.broadcasted_iota(jnp.int32, (128, 128), 0) * 128
           + lax.broadcasted_iota(jnp.int32, (128, 128), 1))

    def body2(_, carry):
        lo2, hi2 = carry
        mid = lax.shift_right_logical(lo2 + hi2, 1)
        cnt = jnp.sum((eq & (lin <= mid)).astype(jnp.int32))
        cond = cnt >= need
        return (jnp.where(cond, lo2, mid + 1), jnp.where(cond, mid, hi2))

    c, _ = lax.fori_loop(0, 14, body2, (jnp.int32(0), jnp.int32(_B - 1)))
    sel = gt | (eq & (lin <= c))

    w = w_ref[...]
    l = l_ref[...]
    num = jnp.sum(jnp.where(sel, w * l, 0.0))
    den = jnp.sum(jnp.where(sel, w, 0.0))
    out_ref[...] = jnp.broadcast_to(num / den, (1, 128))


def kernel(y_hat, y, weights):
    yh_t = y_hat.T
    y_t = y.T
    w2 = weights.reshape(_C, 1)
    slw = pl.pallas_call(
        _row_stats_kernel,
        grid=(_NG,),
        in_specs=[
            pl.BlockSpec((_C, 1), lambda g: (0, 0)),
            pl.BlockSpec((_C, _CB), lambda g: (0, g)),
            pl.BlockSpec((_C, _CB), lambda g: (0, g)),
        ],
        out_specs=pl.BlockSpec((3, _CB), lambda g: (0, g)),
        out_shape=jax.ShapeDtypeStruct((3, _B), jnp.float32),
    )(w2, yh_t, y_t)
    s, l, wr = slw[0], slw[1], slw[2]

    out = pl.pallas_call(
        _select_kernel,
        out_shape=jax.ShapeDtypeStruct((1, 128), jnp.float32),
    )(s.reshape(128, 128), l.reshape(128, 128), wr.reshape(128, 128))
    return out[0, 0]
